# baseline (speedup 1.0000x reference)
import jax
import jax.numpy as jnp
from jax.experimental import pallas as pl
from jax.experimental.pallas import tpu as pltpu

BLOCK_NUM = 256
NSPLIT = 4
BB = 8
SB = BB // NSPLIT


def _sgc_kernel(rmt_ref, *refs):
    adj_refs = refs[:NSPLIT]
    out_ref = refs[NSPLIT]

    def aref(k):
        return adj_refs[k // SB][k % SB]

    bs = [
        jax.lax.dot_general(
            aref(k), aref(k), (((0,), (1,)), ((), ())),
            preferred_element_type=jnp.float32)
        for k in range(BB)
    ]
    us = [
        jnp.dot(rmt_ref[k], bs[k], preferred_element_type=jnp.float32)
        for k in range(BB)
    ]
    fulls = [
        jnp.dot(us[k], bs[k], preferred_element_type=jnp.float32)
        for k in range(BB)
    ]
    diags = [
        jnp.sum(bs[k] * bs[k].T, axis=0, keepdims=True) for k in range(BB)
    ]
    for k in range(BB):
        out_ref[k] = fulls[k] - rmt_ref[k] * diags[k]


def _adj_index_map(j):
    return lambda i: (i * NSPLIT + j, 0, 0)


def kernel(regional_means, adj):
    bsz, n, d = regional_means.shape
    rm_t = jnp.transpose(regional_means, (0, 2, 1))
    adj = pltpu.with_memory_space_constraint(adj, pltpu.MemorySpace.HBM)
    in_specs = [pl.BlockSpec((BB, d, n), lambda i: (i, 0, 0))]
    in_specs += [
        pl.BlockSpec((SB, n, n), _adj_index_map(j)) for j in range(NSPLIT)
    ]
    out_t = pl.pallas_call(
        _sgc_kernel,
        grid=(bsz // BB,),
        in_specs=in_specs,
        out_specs=pl.BlockSpec((BB, d, n), lambda i: (i, 0, 0)),
        out_shape=jax.ShapeDtypeStruct((bsz, d, n), jnp.float32),
    )(rm_t, *([adj] * NSPLIT))
    return jnp.transpose(out_t, (0, 2, 1))
